# baseline (speedup 1.0000x reference)
import functools

import jax
import jax.numpy as jnp
from jax import lax
from jax.experimental import pallas as pl
from jax.experimental.pallas import tpu as pltpu
from jax.experimental.pallas import tpu_sc as plsc

BATCH = 16384
EMBED_DIM = 32
N_ROWS = 1000000
NUM_CORES = 2
NUM_SUBCORES = 16
NUM_WORKERS = NUM_CORES * NUM_SUBCORES
BPW = BATCH // NUM_WORKERS
HALF = BPW // 2
CHUNK = 128
LANES = 16
VROWS = N_ROWS * EMBED_DIM // 128


def _gmf_body(uidx_hbm, iidx_hbm, uemb_hbm, iemb_hbm, out_hbm,
              uraw_v, iraw_v, uvidx_v, ivid_v, uoff_v, ioff_v,
              ubuf_v, ibuf_v, outb_v, sem):
    wid = lax.axis_index("s") * NUM_CORES + lax.axis_index("c")
    base = wid * BPW

    pltpu.sync_copy(uidx_hbm.at[pl.ds(base, BPW)], uraw_v)
    pltpu.sync_copy(iidx_hbm.at[pl.ds(base, BPW)], iraw_v)

    def prep(g, carry):
        sl = pl.ds(g * LANES, LANES)
        u = uraw_v[sl]
        uvidx_v[sl] = u >> 2
        uoff_v[sl] = (u & 3) << 5
        v = iraw_v[sl]
        ivid_v[sl] = v >> 2
        ioff_v[sl] = (v & 3) << 5
        return carry

    lax.fori_loop(0, BPW // LANES, prep, 0)

    iota = lax.iota(jnp.int32, LANES)

    for h in range(2):
        copies = []
        for q in range(2):
            idx_sl = pl.ds((h * 2 + q) * CHUNK, CHUNK)
            buf_sl = pl.ds(q * CHUNK, CHUNK)
            copies.append(pltpu.async_copy(
                uemb_hbm.at[uvidx_v.at[idx_sl]], ubuf_v.at[buf_sl], sem))
            copies.append(pltpu.async_copy(
                iemb_hbm.at[ivid_v.at[idx_sl]], ibuf_v.at[buf_sl], sem))
        for c in copies:
            c.wait()

        def compute(g, carry):
            j0 = h * HALF + g * LANES
            rows = g * LANES + iota
            uoff = uoff_v[pl.ds(j0, LANES)]
            ioff = ioff_v[pl.ds(j0, LANES)]
            tcb = j0 >> 7
            lbase = j0 & 127
            for c in range(EMBED_DIM):
                u16 = plsc.load_gather(ubuf_v, [rows, uoff + c])
                v16 = plsc.load_gather(ibuf_v, [rows, ioff + c])
                outb_v[c // 8, tcb, c % 8, pl.ds(lbase, LANES)] = u16 * v16
            return carry

        lax.fori_loop(0, HALF // LANES, compute, 0)

    for tr in range(4):
        pltpu.sync_copy(outb_v.at[tr],
                        out_hbm.at[tr, pl.ds(wid * 4, 4)])


@jax.jit
def _gmf(uidx, iidx, uemb, iemb):
    mesh = plsc.VectorSubcoreMesh(core_axis_name="c", subcore_axis_name="s")
    run = functools.partial(
        pl.kernel,
        mesh=mesh,
        out_type=jax.ShapeDtypeStruct((4, 128, 8, 128), jnp.float32),
        scratch_types=[
            pltpu.VMEM((BPW,), jnp.int32),
            pltpu.VMEM((BPW,), jnp.int32),
            pltpu.VMEM((BPW,), jnp.int32),
            pltpu.VMEM((BPW,), jnp.int32),
            pltpu.VMEM((BPW,), jnp.int32),
            pltpu.VMEM((BPW,), jnp.int32),
            pltpu.VMEM((HALF, 128), jnp.float32),
            pltpu.VMEM((HALF, 128), jnp.float32),
            pltpu.VMEM((4, 4, 8, 128), jnp.float32),
            pltpu.SemaphoreType.DMA,
        ],
        compiler_params=pltpu.CompilerParams(
            use_tc_tiling_on_sc=True, needs_layout_passes=False),
    )(_gmf_body)
    return run(uidx, iidx, uemb, iemb)


def kernel(user_idx, item_idx, user_emb, item_emb):
    u2 = user_emb.reshape(VROWS, 128)
    v2 = item_emb.reshape(VROWS, 128)
    out4 = _gmf(user_idx.astype(jnp.int32), item_idx.astype(jnp.int32), u2, v2)
    return out4.transpose(1, 3, 0, 2).reshape(BATCH, EMBED_DIM)


# speedup vs baseline: 4.3905x; 4.3905x over previous
import functools

import jax
import jax.numpy as jnp
from jax import lax
from jax.experimental import pallas as pl
from jax.experimental.pallas import tpu as pltpu
from jax.experimental.pallas import tpu_sc as plsc

BATCH = 16384
EMBED_DIM = 32
N_ROWS = 1000000
NUM_CORES = 2
NUM_SUBCORES = 16
NUM_WORKERS = NUM_CORES * NUM_SUBCORES
BPW = BATCH // NUM_WORKERS
LANES = 16
NGROUPS = BPW // LANES
NBUF = 8


def _gmf_body(uidx_hbm, iidx_hbm, uemb_hbm, iemb_hbm, out_hbm,
              uraw_v, iraw_v, ubuf_v, ibuf_v, outb_v, sem):
    wid = lax.axis_index("s") * NUM_CORES + lax.axis_index("c")
    base = wid * BPW

    pltpu.sync_copy(uidx_hbm.at[pl.ds(base, BPW)], uraw_v)
    pltpu.sync_copy(iidx_hbm.at[pl.ds(base, BPW)], iraw_v)

    iota = lax.iota(jnp.int32, LANES)
    tr_lo = iota >> 3
    s_lo = iota & 7
    tr_hi = (iota + LANES) >> 3

    def fire(ru, ri, slot):
        for tr in range(4):
            off_u = pl.multiple_of((ru >> 7) * 128, 128)
            pltpu.async_copy(
                uemb_hbm.at[tr, pl.ds(0, 8), pl.ds(off_u, 128)],
                ubuf_v.at[slot, tr], sem.at[slot])
            off_i = pl.multiple_of((ri >> 7) * 128, 128)
            pltpu.async_copy(
                iemb_hbm.at[tr, pl.ds(0, 8), pl.ds(off_i, 128)],
                ibuf_v.at[slot, tr], sem.at[slot])

    def drain(slot):
        for tr in range(4):
            pltpu.make_async_copy(
                uemb_hbm.at[0, pl.ds(0, 8), pl.ds(0, 128)],
                ubuf_v.at[slot, tr], sem.at[slot]).wait()
            pltpu.make_async_copy(
                iemb_hbm.at[0, pl.ds(0, 8), pl.ds(0, 128)],
                ibuf_v.at[slot, tr], sem.at[slot]).wait()

    head_u = uraw_v[pl.ds(0, LANES)]
    head_i = iraw_v[pl.ds(0, LANES)]
    for k in range(NBUF):
        fire(head_u[k], head_i[k], k)

    def group(g, carry):
        cur_u = uraw_v[pl.ds(g * LANES, LANES)]
        cur_i = iraw_v[pl.ds(g * LANES, LANES)]
        nxt = jnp.minimum((g + 1) * LANES, BPW - LANES)
        nxt_u = uraw_v[pl.ds(nxt, LANES)]
        nxt_i = iraw_v[pl.ds(nxt, LANES)]
        for k in range(LANES):
            j = g * LANES + k
            slot = k % NBUF
            drain(slot)
            ru = cur_u[k]
            ri = cur_i[k]
            lu = lax.broadcast(ru & 127, (LANES,))
            li = lax.broadcast(ri & 127, (LANES,))
            slotv = lax.broadcast(jnp.int32(slot), (LANES,))
            u_lo = plsc.load_gather(ubuf_v, [slotv, tr_lo, s_lo, lu])
            u_hi = plsc.load_gather(ubuf_v, [slotv, tr_hi, s_lo, lu])
            v_lo = plsc.load_gather(ibuf_v, [slotv, tr_lo, s_lo, li])
            v_hi = plsc.load_gather(ibuf_v, [slotv, tr_hi, s_lo, li])
            tcb = lax.broadcast(j >> 7, (LANES,))
            lj = lax.broadcast(j & 127, (LANES,))
            plsc.store_scatter(outb_v, [tr_lo, tcb, s_lo, lj], u_lo * v_lo)
            plsc.store_scatter(outb_v, [tr_hi, tcb, s_lo, lj], u_hi * v_hi)
            if k < LANES - NBUF:
                fire(cur_u[k + NBUF], cur_i[k + NBUF], slot)
            else:
                kk = k + NBUF - LANES

                @pl.when(g < NGROUPS - 1)
                def _():
                    fire(nxt_u[kk], nxt_i[kk], slot)
        return carry

    lax.fori_loop(0, NGROUPS, group, 0)

    for tr in range(4):
        pltpu.sync_copy(outb_v.at[tr], out_hbm.at[tr, pl.ds(wid * 4, 4)])


@jax.jit
def _gmf(uidx, iidx, uemb, iemb):
    mesh = plsc.VectorSubcoreMesh(core_axis_name="c", subcore_axis_name="s")
    run = functools.partial(
        pl.kernel,
        mesh=mesh,
        out_type=jax.ShapeDtypeStruct((4, 128, 8, 128), jnp.float32),
        scratch_types=[
            pltpu.VMEM((BPW,), jnp.int32),
            pltpu.VMEM((BPW,), jnp.int32),
            pltpu.VMEM((NBUF, 4, 8, 128), jnp.float32),
            pltpu.VMEM((NBUF, 4, 8, 128), jnp.float32),
            pltpu.VMEM((4, 4, 8, 128), jnp.float32),
            pltpu.SemaphoreType.DMA((NBUF,)),
        ],
        compiler_params=pltpu.CompilerParams(
            use_tc_tiling_on_sc=True, needs_layout_passes=False),
    )(_gmf_body)
    return run(uidx, iidx, uemb, iemb)


def kernel(user_idx, item_idx, user_emb, item_emb):
    u3 = user_emb.T.reshape(4, 8, N_ROWS)
    v3 = item_emb.T.reshape(4, 8, N_ROWS)
    out4 = _gmf(user_idx.astype(jnp.int32), item_idx.astype(jnp.int32),
                u3, v3)
    return out4.transpose(1, 3, 0, 2).reshape(BATCH, EMBED_DIM)
